# baseline (speedup 1.0000x reference)
import jax
import jax.numpy as jnp
from jax import lax
from jax.experimental import pallas as pl
from jax.experimental.pallas import tpu as pltpu
from jax.experimental.pallas import tpu_sc as plsc

EMB = 64
SCALE = 8.0
NC = 2
NS = 16
NW = NC * NS
CHUNK = 128
LANES = 16


def _body(tokens_hbm, table_hbm, out_hbm, idx_v, rows_v, sem_g):
    wid = lax.axis_index("s") * NC + lax.axis_index("c")
    nch = tokens_hbm.shape[1]
    pltpu.sync_copy(tokens_hbm.at[wid], idx_v)

    def chunk_body(j, carry):
        pltpu.async_copy(table_hbm.at[idx_v.at[j]], rows_v, sem_g).wait()

        def scale_row(r, c2):
            for c in range(EMB // LANES):
                sl = pl.ds(c * LANES, LANES)
                rows_v[r, sl] = rows_v[r, sl] * SCALE
            return c2

        lax.fori_loop(0, CHUNK, scale_row, None)
        pltpu.sync_copy(rows_v, out_hbm.at[wid, j])
        return carry

    lax.fori_loop(0, nch, chunk_body, None)


def kernel(tokens, embedding_weight):
    B, L = tokens.shape
    total = B * L
    assert total % (NW * CHUNK) == 0, total
    nch = total // (NW * CHUNK)
    idx = tokens.reshape(NW, nch, CHUNK).astype(jnp.int32)
    mesh = plsc.VectorSubcoreMesh(core_axis_name="c", subcore_axis_name="s")
    out = pl.kernel(
        _body,
        out_type=jax.ShapeDtypeStruct((NW, nch, CHUNK, EMB), jnp.float32),
        mesh=mesh,
        compiler_params=pltpu.CompilerParams(use_tc_tiling_on_sc=False),
        scratch_types=[
            pltpu.VMEM((nch, CHUNK), jnp.int32),
            pltpu.VMEM((CHUNK, EMB), jnp.float32),
            pltpu.SemaphoreType.DMA,
        ],
    )(idx, embedding_weight)
    return out.reshape(B, L, EMB)


# speedup vs baseline: 1.2096x; 1.2096x over previous
import jax
import jax.numpy as jnp
from jax import lax
from jax.experimental import pallas as pl
from jax.experimental.pallas import tpu as pltpu
from jax.experimental.pallas import tpu_sc as plsc

EMB = 64
SCALE = 8.0
NC = 2
NS = 16
NW = NC * NS
CHUNK = 128
LANES = 16
NBUF = 4


def _body(tokens_hbm, table_hbm, out_hbm, idx_v, rows_v, *sems):
    sem_g = sems[:NBUF]
    sem_w = sems[NBUF:]
    wid = lax.axis_index("s") * NC + lax.axis_index("c")
    nch = tokens_hbm.shape[1]
    pltpu.sync_copy(tokens_hbm.at[wid], idx_v)

    for b in range(NBUF):
        pltpu.async_copy(table_hbm.at[idx_v.at[b]], rows_v.at[b], sem_g[b])

    def group(g, carry):
        for b in range(NBUF):
            j = g * NBUF + b
            pb = (b - 1) % NBUF
            jn = j - 1 + NBUF

            @pl.when(jnp.logical_and(j >= 1, jn < nch))
            def _():
                pltpu.make_async_copy(
                    rows_v.at[pb], out_hbm.at[wid, 0], sem_w[pb]).wait()
                pltpu.async_copy(
                    table_hbm.at[idx_v.at[jn]], rows_v.at[pb], sem_g[pb])

            pltpu.make_async_copy(
                table_hbm.at[idx_v.at[j]], rows_v.at[b], sem_g[b]).wait()

            @plsc.parallel_loop(0, CHUNK, unroll=4)
            def _scale(r):
                for c in range(EMB // LANES):
                    sl = pl.ds(c * LANES, LANES)
                    rows_v[b, r, sl] = rows_v[b, r, sl] * SCALE

            pltpu.async_copy(rows_v.at[b], out_hbm.at[wid, j], sem_w[b])
        return carry

    lax.fori_loop(0, nch // NBUF, group, None)

    for b in range(NBUF):
        pltpu.make_async_copy(rows_v.at[b], out_hbm.at[wid, 0], sem_w[b]).wait()


def kernel(tokens, embedding_weight):
    B, L = tokens.shape
    total = B * L
    assert total % (NW * CHUNK * NBUF) == 0, total
    nch = total // (NW * CHUNK)
    idx = tokens.reshape(NW, nch, CHUNK).astype(jnp.int32)
    mesh = plsc.VectorSubcoreMesh(core_axis_name="c", subcore_axis_name="s")
    out = pl.kernel(
        _body,
        out_type=jax.ShapeDtypeStruct((NW, nch, CHUNK, EMB), jnp.float32),
        mesh=mesh,
        compiler_params=pltpu.CompilerParams(use_tc_tiling_on_sc=False),
        scratch_types=[
            pltpu.VMEM((nch, CHUNK), jnp.int32),
            pltpu.VMEM((NBUF, CHUNK, EMB), jnp.float32),
        ] + [pltpu.SemaphoreType.DMA] * (2 * NBUF),
    )(idx, embedding_weight)
    return out.reshape(B, L, EMB)
